# baseline (speedup 1.0000x reference)
import functools

import jax
import jax.numpy as jnp
from jax import lax
from jax.experimental import pallas as pl
from jax.experimental.pallas import tpu as pltpu
from jax.experimental.pallas import tpu_sc as plsc

_CHUNK = 128
_NW = 32


def _sc_aggregate(table, src2, dst2, npad):
    n, d = table.shape
    n_rows = src2.shape[0]
    total = n_rows // 16
    cpt0 = int(round(total * 0.614))
    cpt1 = total - cpt0
    assert cpt0 > 0 and cpt1 > 0
    mesh = plsc.VectorSubcoreMesh(core_axis_name="c", subcore_axis_name="s")
    rpt = npad // 16

    @functools.partial(
        pl.kernel,
        mesh=mesh,
        out_type=jax.ShapeDtypeStruct((2, npad, d), jnp.float32),
        scratch_types=[
            pltpu.VMEM((_CHUNK,), jnp.int32),
            pltpu.VMEM((_CHUNK,), jnp.int32),
            pltpu.VMEM((_CHUNK, d), jnp.float32),
            pltpu.VMEM_SHARED((npad, d), jnp.float32),
            pltpu.SemaphoreType.DMA,
        ],
    )
    def agg_kernel(table_hbm, src_hbm, dst_hbm, zeros_hbm, out_hbm,
                   src_v, dst_v, rows_v, acc_sh, sem_r):
        cid = lax.axis_index("c")
        sid = lax.axis_index("s")

        pltpu.sync_copy(zeros_hbm.at[pl.ds(sid * rpt, rpt)],
                        acc_sh.at[pl.ds(sid * rpt, rpt)])
        plsc.subcore_barrier()

        nch = jnp.where(cid == 0, cpt0, cpt1)
        row0 = jnp.where(cid == 0, sid * cpt0, 16 * cpt0 + sid * cpt1)

        def step(i, carry):
            pltpu.sync_copy(src_hbm.at[row0 + i], src_v)
            pltpu.sync_copy(dst_hbm.at[row0 + i], dst_v)
            pltpu.async_copy(table_hbm.at[src_v], rows_v, sem_r).wait()
            pltpu.sync_copy(rows_v, acc_sh.at[dst_v], add=True)
            return carry

        lax.fori_loop(0, nch, step, 0)
        plsc.subcore_barrier()

        pltpu.sync_copy(acc_sh.at[pl.ds(sid * rpt, rpt)],
                        out_hbm.at[cid, pl.ds(sid * rpt, rpt)])

    zeros = jnp.zeros((npad, d), jnp.float32)
    return agg_kernel(table, src2, dst2, zeros)


def _tc_dense(parts, x_cur, onehot, hw_t, hb, ww_t, wb, blk, fp_prev):
    n, d = x_cur.shape
    fp = ww_t.shape[1]
    g = onehot.shape[1]
    grid = n // blk
    have_prev = fp_prev is not None
    if not have_prev:
        fp_prev = jnp.zeros((1, 1), jnp.float32)

    def body(parts_ref, x_ref, oh_ref, hwt_ref, hb_ref, wwt_ref, wb_ref,
             fpp_ref, upd_ref, fp_ref):
        agg = parts_ref[0] + parts_ref[1] + x_ref[...]
        pre = jnp.dot(agg, hwt_ref[...], preferred_element_type=jnp.float32)
        pre = pre + hb_ref[...]
        upd = 1.0 / (1.0 + jnp.exp(-pre))
        upd_ref[...] = upd
        logits = jnp.dot(upd, wwt_ref[...], preferred_element_type=jnp.float32)
        logits = logits + wb_ref[...]
        m = jnp.max(logits, axis=-1, keepdims=True)
        e = jnp.exp(logits - m)
        p = e / jnp.sum(e, axis=-1, keepdims=True)
        contrib = lax.dot_general(oh_ref[...], p, (((0,), (0,)), ((), ())),
                                  preferred_element_type=jnp.float32)

        @pl.when(pl.program_id(0) == 0)
        def _init():
            if have_prev:
                fp_ref[...] = fpp_ref[...] + contrib
            else:
                fp_ref[...] = contrib

        @pl.when(pl.program_id(0) != 0)
        def _acc():
            fp_ref[...] += contrib

    fpp_spec = (pl.BlockSpec((g, fp), lambda i: (0, 0)) if have_prev
                else pl.BlockSpec((1, 1), lambda i: (0, 0)))
    return pl.pallas_call(
        body,
        grid=(grid,),
        in_specs=[
            pl.BlockSpec((2, blk, d), lambda i: (0, i, 0)),
            pl.BlockSpec((blk, d), lambda i: (i, 0)),
            pl.BlockSpec((blk, g), lambda i: (i, 0)),
            pl.BlockSpec((d, d), lambda i: (0, 0)),
            pl.BlockSpec((1, d), lambda i: (0, 0)),
            pl.BlockSpec((d, fp), lambda i: (0, 0)),
            pl.BlockSpec((1, fp), lambda i: (0, 0)),
            fpp_spec,
        ],
        out_specs=[
            pl.BlockSpec((blk, d), lambda i: (i, 0)),
            pl.BlockSpec((g, fp), lambda i: (0, 0)),
        ],
        out_shape=[
            jax.ShapeDtypeStruct((n, d), jnp.float32),
            jax.ShapeDtypeStruct((g, fp), jnp.float32),
        ],
    )(parts, x_cur, onehot, hw_t, hb, ww_t, wb, fp_prev)


def kernel(x, edge_index, batch, H1_w, H1_b, W1_w, W1_b, H2_w, H2_b, W2_w, W2_b):
    n, d = x.shape
    fp = W1_w.shape[0]
    g = 64
    e = edge_index.shape[1]
    npad = ((n + 1 + 255) // 256) * 256

    q = 32 * _CHUNK
    ep = ((e + q - 1) // q) * q
    pad = ep - e
    src = jnp.concatenate(
        [edge_index[0].astype(jnp.int32), jnp.zeros((pad,), jnp.int32)])
    dst = jnp.concatenate(
        [edge_index[1].astype(jnp.int32), jnp.full((pad,), n, jnp.int32)])
    src = src.reshape(-1, _CHUNK)
    dst = dst.reshape(-1, _CHUNK)

    onehot = (batch[:, None] == jnp.arange(g, dtype=batch.dtype)[None, :])
    onehot = onehot.astype(jnp.float32)
    h1t, w1t = H1_w.T, W1_w.T
    h2t, w2t = H2_w.T, W2_w.T
    h1b, w1b = H1_b.reshape(1, d), W1_b.reshape(1, fp)
    h2b, w2b = H2_b.reshape(1, d), W2_b.reshape(1, fp)

    blk = 400
    parts1 = _sc_aggregate(x, src, dst, npad)
    upd1, fp1 = _tc_dense(parts1, x, onehot, h1t, h1b, w1t, w1b, blk, None)
    parts2 = _sc_aggregate(upd1, src, dst, npad)
    _, out = _tc_dense(parts2, upd1, onehot, h2t, h2b, w2t, w2b, blk, fp1)
    return out


# speedup vs baseline: 1.0181x; 1.0181x over previous
import functools

import jax
import jax.numpy as jnp
from jax import lax
from jax.experimental import pallas as pl
from jax.experimental.pallas import tpu as pltpu
from jax.experimental.pallas import tpu_sc as plsc

_CHUNK = 128
_NW = 32


def _sc_aggregate(table, src2, dst2, npad):
    n, d = table.shape
    n_rows = src2.shape[0]
    total = n_rows // 16
    cpt0 = int(round(total * 0.614))
    cpt1 = total - cpt0
    assert cpt0 > 0 and cpt1 > 0
    mesh = plsc.VectorSubcoreMesh(core_axis_name="c", subcore_axis_name="s")
    rpt = npad // 16

    @functools.partial(
        pl.kernel,
        mesh=mesh,
        out_type=jax.ShapeDtypeStruct((2, npad, d), jnp.float32),
        scratch_types=[
            pltpu.VMEM((_CHUNK,), jnp.int32),
            pltpu.VMEM((_CHUNK,), jnp.int32),
            pltpu.VMEM((_CHUNK, d), jnp.float32),
            pltpu.VMEM_SHARED((npad, d), jnp.float32),
            pltpu.SemaphoreType.DMA,
        ],
    )
    def agg_kernel(table_hbm, src_hbm, dst_hbm, zeros_hbm, out_hbm,
                   src_v, dst_v, rows_v, acc_sh, sem_r):
        cid = lax.axis_index("c")
        sid = lax.axis_index("s")

        pltpu.sync_copy(zeros_hbm.at[pl.ds(sid * rpt, rpt)],
                        acc_sh.at[pl.ds(sid * rpt, rpt)])
        plsc.subcore_barrier()

        nch = jnp.where(cid == 0, cpt0, cpt1)
        row0 = jnp.where(cid == 0, sid * cpt0, 16 * cpt0 + sid * cpt1)

        def step(i, carry):
            pltpu.sync_copy(src_hbm.at[row0 + i], src_v)
            pltpu.sync_copy(dst_hbm.at[row0 + i], dst_v)
            pltpu.async_copy(table_hbm.at[src_v], rows_v, sem_r).wait()
            pltpu.sync_copy(rows_v, acc_sh.at[dst_v], add=True)
            return carry

        lax.fori_loop(0, nch, step, 0)
        plsc.subcore_barrier()

        pltpu.sync_copy(acc_sh.at[pl.ds(sid * rpt, rpt)],
                        out_hbm.at[cid, pl.ds(sid * rpt, rpt)])

    zeros = jnp.zeros((npad, d), jnp.float32)
    return agg_kernel(table, src2, dst2, zeros)


def _tc_dense(parts, x_cur, onehot, hw_t, hb, ww_t, wb, blk):
    n, d = x_cur.shape
    fp = ww_t.shape[1]
    g = onehot.shape[1]
    grid = n // blk
    def body(parts_ref, x_ref, oh_ref, hwt_ref, hb_ref, wwt_ref, wb_ref,
             upd_ref, fp_ref):
        agg = parts_ref[0] + parts_ref[1] + x_ref[...]
        pre = jnp.dot(agg, hwt_ref[...], preferred_element_type=jnp.float32)
        pre = pre + hb_ref[...]
        upd = 1.0 / (1.0 + jnp.exp(-pre))
        upd_ref[...] = upd
        logits = jnp.dot(upd, wwt_ref[...], preferred_element_type=jnp.float32)
        logits = logits + wb_ref[...]
        m = jnp.max(logits, axis=-1, keepdims=True)
        e = jnp.exp(logits - m)
        p = e / jnp.sum(e, axis=-1, keepdims=True)
        contrib = lax.dot_general(oh_ref[...], p, (((0,), (0,)), ((), ())),
                                  preferred_element_type=jnp.float32)

        @pl.when(pl.program_id(0) == 0)
        def _init():
            fp_ref[...] = contrib

        @pl.when(pl.program_id(0) != 0)
        def _acc():
            fp_ref[...] += contrib

    return pl.pallas_call(
        body,
        grid=(grid,),
        in_specs=[
            pl.BlockSpec((2, blk, d), lambda i: (0, i, 0)),
            pl.BlockSpec((blk, d), lambda i: (i, 0)),
            pl.BlockSpec((blk, g), lambda i: (i, 0)),
            pl.BlockSpec((d, d), lambda i: (0, 0)),
            pl.BlockSpec((1, d), lambda i: (0, 0)),
            pl.BlockSpec((d, fp), lambda i: (0, 0)),
            pl.BlockSpec((1, fp), lambda i: (0, 0)),
        ],
        out_specs=[
            pl.BlockSpec((blk, d), lambda i: (i, 0)),
            pl.BlockSpec((g, fp), lambda i: (0, 0)),
        ],
        out_shape=[
            jax.ShapeDtypeStruct((n, d), jnp.float32),
            jax.ShapeDtypeStruct((g, fp), jnp.float32),
        ],
    )(parts, x_cur, onehot, hw_t, hb, ww_t, wb)


def kernel(x, edge_index, batch, H1_w, H1_b, W1_w, W1_b, H2_w, H2_b, W2_w, W2_b):
    n, d = x.shape
    fp = W1_w.shape[0]
    g = 64
    e = edge_index.shape[1]
    npad = ((n + 1 + 255) // 256) * 256

    q = 32 * _CHUNK
    ep = ((e + q - 1) // q) * q
    pad = ep - e
    src = jnp.concatenate(
        [edge_index[0].astype(jnp.int32), jnp.zeros((pad,), jnp.int32)])
    dst = jnp.concatenate(
        [edge_index[1].astype(jnp.int32), jnp.full((pad,), n, jnp.int32)])
    src = src.reshape(-1, _CHUNK)
    dst = dst.reshape(-1, _CHUNK)

    onehot = (batch[:, None] == jnp.arange(g, dtype=batch.dtype)[None, :])
    onehot = onehot.astype(jnp.float32)
    h1t, w1t = H1_w.T, W1_w.T
    h2t, w2t = H2_w.T, W2_w.T
    h1b, w1b = H1_b.reshape(1, d), W1_b.reshape(1, fp)
    h2b, w2b = H2_b.reshape(1, d), W2_b.reshape(1, fp)

    blk = 400
    parts1 = _sc_aggregate(x, src, dst, npad)
    upd1, fp1 = _tc_dense(parts1, x, onehot, h1t, h1b, w1t, w1b, blk)
    parts2 = _sc_aggregate(upd1, src, dst, npad)
    _, fp2 = _tc_dense(parts2, upd1, onehot, h2t, h2b, w2t, w2b, blk)
    return fp1 + fp2


# speedup vs baseline: 1.0370x; 1.0186x over previous
import functools

import jax
import jax.numpy as jnp
from jax import lax
from jax.experimental import pallas as pl
from jax.experimental.pallas import tpu as pltpu
from jax.experimental.pallas import tpu_sc as plsc

_CHUNK = 128
_NW = 32


def _sc_aggregate(table, src2, dst2, npad):
    n, d = table.shape
    n_rows = src2.shape[0]
    total = n_rows // 16
    cpt0 = int(round(total * 0.63))
    cpt1 = total - cpt0
    assert cpt0 > 0 and cpt1 > 0
    mesh = plsc.VectorSubcoreMesh(core_axis_name="c", subcore_axis_name="s")
    rpt = npad // 16

    @functools.partial(
        pl.kernel,
        mesh=mesh,
        out_type=jax.ShapeDtypeStruct((2, npad, d), jnp.float32),
        scratch_types=[
            pltpu.VMEM((_CHUNK,), jnp.int32),
            pltpu.VMEM((_CHUNK,), jnp.int32),
            pltpu.VMEM((_CHUNK, d), jnp.float32),
            pltpu.VMEM_SHARED((npad, d), jnp.float32),
            pltpu.SemaphoreType.DMA,
        ],
    )
    def agg_kernel(table_hbm, src_hbm, dst_hbm, zeros_hbm, out_hbm,
                   src_v, dst_v, rows_v, acc_sh, sem_r):
        cid = lax.axis_index("c")
        sid = lax.axis_index("s")

        pltpu.sync_copy(zeros_hbm.at[pl.ds(sid * rpt, rpt)],
                        acc_sh.at[pl.ds(sid * rpt, rpt)])
        plsc.subcore_barrier()

        nch = jnp.where(cid == 0, cpt0, cpt1)
        row0 = jnp.where(cid == 0, sid * cpt0, 16 * cpt0 + sid * cpt1)

        def step(i, carry):
            pltpu.sync_copy(src_hbm.at[row0 + i], src_v)
            pltpu.sync_copy(dst_hbm.at[row0 + i], dst_v)
            pltpu.async_copy(table_hbm.at[src_v], rows_v, sem_r).wait()
            pltpu.sync_copy(rows_v, acc_sh.at[dst_v], add=True)
            return carry

        lax.fori_loop(0, nch, step, 0)
        plsc.subcore_barrier()

        pltpu.sync_copy(acc_sh.at[pl.ds(sid * rpt, rpt)],
                        out_hbm.at[cid, pl.ds(sid * rpt, rpt)])

    zeros = jnp.zeros((npad, d), jnp.float32)
    return agg_kernel(table, src2, dst2, zeros)


def _tc_dense(parts, x_cur, onehot, hw_t, hb, ww_t, wb, blk):
    n, d = x_cur.shape
    fp = ww_t.shape[1]
    g = onehot.shape[1]
    grid = n // blk
    def body(parts_ref, x_ref, oh_ref, hwt_ref, hb_ref, wwt_ref, wb_ref,
             upd_ref, fp_ref):
        agg = parts_ref[0] + parts_ref[1] + x_ref[...]
        pre = jnp.dot(agg, hwt_ref[...], preferred_element_type=jnp.float32)
        pre = pre + hb_ref[...]
        upd = 1.0 / (1.0 + jnp.exp(-pre))
        upd_ref[...] = upd
        logits = jnp.dot(upd, wwt_ref[...], preferred_element_type=jnp.float32)
        logits = logits + wb_ref[...]
        m = jnp.max(logits, axis=-1, keepdims=True)
        e = jnp.exp(logits - m)
        p = e / jnp.sum(e, axis=-1, keepdims=True)
        contrib = lax.dot_general(oh_ref[...], p, (((0,), (0,)), ((), ())),
                                  preferred_element_type=jnp.float32)

        @pl.when(pl.program_id(0) == 0)
        def _init():
            fp_ref[...] = contrib

        @pl.when(pl.program_id(0) != 0)
        def _acc():
            fp_ref[...] += contrib

    return pl.pallas_call(
        body,
        grid=(grid,),
        in_specs=[
            pl.BlockSpec((2, blk, d), lambda i: (0, i, 0)),
            pl.BlockSpec((blk, d), lambda i: (i, 0)),
            pl.BlockSpec((blk, g), lambda i: (i, 0)),
            pl.BlockSpec((d, d), lambda i: (0, 0)),
            pl.BlockSpec((1, d), lambda i: (0, 0)),
            pl.BlockSpec((d, fp), lambda i: (0, 0)),
            pl.BlockSpec((1, fp), lambda i: (0, 0)),
        ],
        out_specs=[
            pl.BlockSpec((blk, d), lambda i: (i, 0)),
            pl.BlockSpec((g, fp), lambda i: (0, 0)),
        ],
        out_shape=[
            jax.ShapeDtypeStruct((n, d), jnp.float32),
            jax.ShapeDtypeStruct((g, fp), jnp.float32),
        ],
    )(parts, x_cur, onehot, hw_t, hb, ww_t, wb)


def kernel(x, edge_index, batch, H1_w, H1_b, W1_w, W1_b, H2_w, H2_b, W2_w, W2_b):
    n, d = x.shape
    fp = W1_w.shape[0]
    g = 64
    e = edge_index.shape[1]
    npad = ((n + 1 + 255) // 256) * 256

    q = 32 * _CHUNK
    ep = ((e + q - 1) // q) * q
    pad = ep - e
    src = jnp.concatenate(
        [edge_index[0].astype(jnp.int32), jnp.zeros((pad,), jnp.int32)])
    dst = jnp.concatenate(
        [edge_index[1].astype(jnp.int32), jnp.full((pad,), n, jnp.int32)])
    src = src.reshape(-1, _CHUNK)
    dst = dst.reshape(-1, _CHUNK)

    onehot = (batch[:, None] == jnp.arange(g, dtype=batch.dtype)[None, :])
    onehot = onehot.astype(jnp.float32)
    h1t, w1t = H1_w.T, W1_w.T
    h2t, w2t = H2_w.T, W2_w.T
    h1b, w1b = H1_b.reshape(1, d), W1_b.reshape(1, fp)
    h2b, w2b = H2_b.reshape(1, d), W2_b.reshape(1, fp)

    blk = 400
    parts1 = _sc_aggregate(x, src, dst, npad)
    upd1, fp1 = _tc_dense(parts1, x, onehot, h1t, h1b, w1t, w1b, blk)
    parts2 = _sc_aggregate(upd1, src, dst, npad)
    _, fp2 = _tc_dense(parts2, upd1, onehot, h2t, h2b, w2t, w2b, blk)
    return fp1 + fp2


# speedup vs baseline: 1.1730x; 1.1311x over previous
import functools

import jax
import jax.numpy as jnp
from jax import lax
from jax.experimental import pallas as pl
from jax.experimental.pallas import tpu as pltpu
from jax.experimental.pallas import tpu_sc as plsc

_CHUNK = 128
_NW = 32


def _sc_aggregate(table, src2, dst2, npad):
    n, d = table.shape
    n_rows = src2.shape[0]
    total = n_rows // 16
    cpt0 = int(round(total * 0.63)) & ~1
    cpt1 = total - cpt0
    assert cpt0 > 0 and cpt1 > 0 and cpt1 % 2 == 0
    mesh = plsc.VectorSubcoreMesh(core_axis_name="c", subcore_axis_name="s")
    rpt = npad // 16

    @functools.partial(
        pl.kernel,
        mesh=mesh,
        out_type=jax.ShapeDtypeStruct((2, npad, d), jnp.float32),
        scratch_types=[
            pltpu.VMEM((_CHUNK,), jnp.int32),
            pltpu.VMEM((_CHUNK,), jnp.int32),
            pltpu.VMEM((_CHUNK,), jnp.int32),
            pltpu.VMEM((_CHUNK,), jnp.int32),
            pltpu.VMEM((_CHUNK, d), jnp.float32),
            pltpu.VMEM_SHARED((npad, d), jnp.float32),
            pltpu.SemaphoreType.DMA,
            pltpu.SemaphoreType.DMA,
            pltpu.SemaphoreType.DMA,
        ],
    )
    def agg_kernel(table_hbm, src_hbm, dst_hbm, zeros_hbm, out_hbm,
                   src_a, dst_a, src_b, dst_b, rows_v, acc_sh,
                   sem_a, sem_b, sem_r):
        cid = lax.axis_index("c")
        sid = lax.axis_index("s")

        pltpu.sync_copy(zeros_hbm.at[pl.ds(sid * rpt, rpt)],
                        acc_sh.at[pl.ds(sid * rpt, rpt)])
        plsc.subcore_barrier()

        npairs = jnp.where(cid == 0, cpt0 // 2, cpt1 // 2)
        row0 = jnp.where(cid == 0, sid * cpt0, 16 * cpt0 + sid * cpt1)

        pltpu.async_copy(src_hbm.at[row0], src_a, sem_a)
        pltpu.async_copy(dst_hbm.at[row0], dst_a, sem_a)

        def pair(i, carry):
            j = row0 + 2 * i
            pltpu.make_async_copy(src_hbm.at[j], src_a, sem_a).wait()
            pltpu.make_async_copy(dst_hbm.at[j], dst_a, sem_a).wait()
            pltpu.async_copy(src_hbm.at[j + 1], src_b, sem_b)
            pltpu.async_copy(dst_hbm.at[j + 1], dst_b, sem_b)
            pltpu.async_copy(table_hbm.at[src_a], rows_v, sem_r).wait()
            pltpu.sync_copy(rows_v, acc_sh.at[dst_a], add=True)
            pltpu.make_async_copy(src_hbm.at[j + 1], src_b, sem_b).wait()
            pltpu.make_async_copy(dst_hbm.at[j + 1], dst_b, sem_b).wait()

            @pl.when(i < npairs - 1)
            def _pre():
                pltpu.async_copy(src_hbm.at[j + 2], src_a, sem_a)
                pltpu.async_copy(dst_hbm.at[j + 2], dst_a, sem_a)

            pltpu.async_copy(table_hbm.at[src_b], rows_v, sem_r).wait()
            pltpu.sync_copy(rows_v, acc_sh.at[dst_b], add=True)
            return carry

        lax.fori_loop(0, npairs, pair, 0)
        plsc.subcore_barrier()

        pltpu.sync_copy(acc_sh.at[pl.ds(sid * rpt, rpt)],
                        out_hbm.at[cid, pl.ds(sid * rpt, rpt)])

    zeros = jnp.zeros((npad, d), jnp.float32)
    return agg_kernel(table, src2, dst2, zeros)


def _tc_dense(parts, x_cur, onehot, hw_t, hb, ww_t, wb, blk):
    n, d = x_cur.shape
    fp = ww_t.shape[1]
    g = onehot.shape[1]
    grid = n // blk
    def body(parts_ref, x_ref, oh_ref, hwt_ref, hb_ref, wwt_ref, wb_ref,
             upd_ref, fp_ref):
        agg = parts_ref[0] + parts_ref[1] + x_ref[...]
        pre = jnp.dot(agg, hwt_ref[...], preferred_element_type=jnp.float32)
        pre = pre + hb_ref[...]
        upd = 1.0 / (1.0 + jnp.exp(-pre))
        upd_ref[...] = upd
        logits = jnp.dot(upd, wwt_ref[...], preferred_element_type=jnp.float32)
        logits = logits + wb_ref[...]
        m = jnp.max(logits, axis=-1, keepdims=True)
        e = jnp.exp(logits - m)
        p = e / jnp.sum(e, axis=-1, keepdims=True)
        contrib = lax.dot_general(oh_ref[...], p, (((0,), (0,)), ((), ())),
                                  preferred_element_type=jnp.float32)

        @pl.when(pl.program_id(0) == 0)
        def _init():
            fp_ref[...] = contrib

        @pl.when(pl.program_id(0) != 0)
        def _acc():
            fp_ref[...] += contrib

    return pl.pallas_call(
        body,
        grid=(grid,),
        in_specs=[
            pl.BlockSpec((2, blk, d), lambda i: (0, i, 0)),
            pl.BlockSpec((blk, d), lambda i: (i, 0)),
            pl.BlockSpec((blk, g), lambda i: (i, 0)),
            pl.BlockSpec((d, d), lambda i: (0, 0)),
            pl.BlockSpec((1, d), lambda i: (0, 0)),
            pl.BlockSpec((d, fp), lambda i: (0, 0)),
            pl.BlockSpec((1, fp), lambda i: (0, 0)),
        ],
        out_specs=[
            pl.BlockSpec((blk, d), lambda i: (i, 0)),
            pl.BlockSpec((g, fp), lambda i: (0, 0)),
        ],
        out_shape=[
            jax.ShapeDtypeStruct((n, d), jnp.float32),
            jax.ShapeDtypeStruct((g, fp), jnp.float32),
        ],
    )(parts, x_cur, onehot, hw_t, hb, ww_t, wb)


def kernel(x, edge_index, batch, H1_w, H1_b, W1_w, W1_b, H2_w, H2_b, W2_w, W2_b):
    n, d = x.shape
    fp = W1_w.shape[0]
    g = 64
    e = edge_index.shape[1]
    npad = ((n + 1 + 255) // 256) * 256

    q = 32 * _CHUNK
    ep = ((e + q - 1) // q) * q
    pad = ep - e
    src = jnp.concatenate(
        [edge_index[0].astype(jnp.int32), jnp.zeros((pad,), jnp.int32)])
    dst = jnp.concatenate(
        [edge_index[1].astype(jnp.int32), jnp.full((pad,), n, jnp.int32)])
    src = src.reshape(-1, _CHUNK)
    dst = dst.reshape(-1, _CHUNK)

    onehot = (batch[:, None] == jnp.arange(g, dtype=batch.dtype)[None, :])
    onehot = onehot.astype(jnp.float32)
    h1t, w1t = H1_w.T, W1_w.T
    h2t, w2t = H2_w.T, W2_w.T
    h1b, w1b = H1_b.reshape(1, d), W1_b.reshape(1, fp)
    h2b, w2b = H2_b.reshape(1, d), W2_b.reshape(1, fp)

    blk = 400
    parts1 = _sc_aggregate(x, src, dst, npad)
    upd1, fp1 = _tc_dense(parts1, x, onehot, h1t, h1b, w1t, w1b, blk)
    parts2 = _sc_aggregate(upd1, src, dst, npad)
    _, fp2 = _tc_dense(parts2, upd1, onehot, h2t, h2b, w2t, w2b, blk)
    return fp1 + fp2


# speedup vs baseline: 1.2712x; 1.0837x over previous
import functools

import jax
import jax.numpy as jnp
from jax import lax
from jax.experimental import pallas as pl
from jax.experimental.pallas import tpu as pltpu
from jax.experimental.pallas import tpu_sc as plsc

_CHUNK = 128
_NW = 32


def _sc_aggregate(table, src2, dst2, npad):
    n, d = table.shape
    n_rows = src2.shape[0]
    total = n_rows // 16
    cpt0 = int(round(total * 0.63)) & ~1
    cpt1 = total - cpt0
    assert cpt0 > 0 and cpt1 > 0 and cpt1 % 2 == 0
    mesh = plsc.VectorSubcoreMesh(core_axis_name="c", subcore_axis_name="s")
    rpt = npad // 16

    @functools.partial(
        pl.kernel,
        mesh=mesh,
        out_type=jax.ShapeDtypeStruct((2, npad, d), jnp.float32),
        scratch_types=[
            pltpu.VMEM((_CHUNK,), jnp.int32),
            pltpu.VMEM((_CHUNK,), jnp.int32),
            pltpu.VMEM((_CHUNK,), jnp.int32),
            pltpu.VMEM((_CHUNK,), jnp.int32),
            pltpu.VMEM((_CHUNK, d), jnp.float32),
            pltpu.VMEM((_CHUNK, d), jnp.float32),
            pltpu.VMEM_SHARED((npad, d), jnp.float32),
            pltpu.SemaphoreType.DMA,
            pltpu.SemaphoreType.DMA,
            pltpu.SemaphoreType.DMA,
            pltpu.SemaphoreType.DMA,
        ],
    )
    def agg_kernel(table_hbm, src_hbm, dst_hbm, zeros_hbm, out_hbm,
                   src_a, dst_a, src_b, dst_b, rows_a, rows_b, acc_sh,
                   sem_a, sem_b, sem_ra, sem_rb):
        cid = lax.axis_index("c")
        sid = lax.axis_index("s")

        pltpu.sync_copy(zeros_hbm.at[pl.ds(sid * rpt, rpt)],
                        acc_sh.at[pl.ds(sid * rpt, rpt)])
        plsc.subcore_barrier()

        npairs = jnp.where(cid == 0, cpt0 // 2, cpt1 // 2)
        row0 = jnp.where(cid == 0, sid * cpt0, 16 * cpt0 + sid * cpt1)

        pltpu.async_copy(src_hbm.at[row0], src_a, sem_a)
        pltpu.async_copy(dst_hbm.at[row0], dst_a, sem_a)
        pltpu.async_copy(src_hbm.at[row0 + 1], src_b, sem_b)
        pltpu.async_copy(dst_hbm.at[row0 + 1], dst_b, sem_b)
        pltpu.make_async_copy(src_hbm.at[row0], src_a, sem_a).wait()
        pltpu.make_async_copy(dst_hbm.at[row0], dst_a, sem_a).wait()
        pltpu.async_copy(table_hbm.at[src_a], rows_a, sem_ra)

        def pair(i, carry):
            j = row0 + 2 * i
            not_last = i < npairs - 1
            pltpu.make_async_copy(table_hbm.at[src_a], rows_a, sem_ra).wait()
            pltpu.make_async_copy(src_hbm.at[j + 1], src_b, sem_b).wait()
            pltpu.make_async_copy(dst_hbm.at[j + 1], dst_b, sem_b).wait()
            pltpu.async_copy(table_hbm.at[src_b], rows_b, sem_rb)
            pltpu.sync_copy(rows_a, acc_sh.at[dst_a], add=True)

            @pl.when(not_last)
            def _pre_a():
                pltpu.async_copy(src_hbm.at[j + 2], src_a, sem_a)
                pltpu.async_copy(dst_hbm.at[j + 2], dst_a, sem_a)

            pltpu.make_async_copy(table_hbm.at[src_b], rows_b, sem_rb).wait()

            @pl.when(not_last)
            def _next_a():
                pltpu.make_async_copy(src_hbm.at[j + 2], src_a, sem_a).wait()
                pltpu.make_async_copy(dst_hbm.at[j + 2], dst_a, sem_a).wait()
                pltpu.async_copy(table_hbm.at[src_a], rows_a, sem_ra)

            pltpu.sync_copy(rows_b, acc_sh.at[dst_b], add=True)

            @pl.when(not_last)
            def _pre_b():
                pltpu.async_copy(src_hbm.at[j + 3], src_b, sem_b)
                pltpu.async_copy(dst_hbm.at[j + 3], dst_b, sem_b)

            return carry

        lax.fori_loop(0, npairs, pair, 0)
        plsc.subcore_barrier()

        pltpu.sync_copy(acc_sh.at[pl.ds(sid * rpt, rpt)],
                        out_hbm.at[cid, pl.ds(sid * rpt, rpt)])

    zeros = jnp.zeros((npad, d), jnp.float32)
    return agg_kernel(table, src2, dst2, zeros)


def _tc_dense(parts, x_cur, onehot, hw_t, hb, ww_t, wb, blk):
    n, d = x_cur.shape
    fp = ww_t.shape[1]
    g = onehot.shape[1]
    grid = n // blk
    def body(parts_ref, x_ref, oh_ref, hwt_ref, hb_ref, wwt_ref, wb_ref,
             upd_ref, fp_ref):
        agg = parts_ref[0] + parts_ref[1] + x_ref[...]
        pre = jnp.dot(agg, hwt_ref[...], preferred_element_type=jnp.float32)
        pre = pre + hb_ref[...]
        upd = 1.0 / (1.0 + jnp.exp(-pre))
        upd_ref[...] = upd
        logits = jnp.dot(upd, wwt_ref[...], preferred_element_type=jnp.float32)
        logits = logits + wb_ref[...]
        m = jnp.max(logits, axis=-1, keepdims=True)
        e = jnp.exp(logits - m)
        p = e / jnp.sum(e, axis=-1, keepdims=True)
        contrib = lax.dot_general(oh_ref[...], p, (((0,), (0,)), ((), ())),
                                  preferred_element_type=jnp.float32)

        @pl.when(pl.program_id(0) == 0)
        def _init():
            fp_ref[...] = contrib

        @pl.when(pl.program_id(0) != 0)
        def _acc():
            fp_ref[...] += contrib

    return pl.pallas_call(
        body,
        grid=(grid,),
        in_specs=[
            pl.BlockSpec((2, blk, d), lambda i: (0, i, 0)),
            pl.BlockSpec((blk, d), lambda i: (i, 0)),
            pl.BlockSpec((blk, g), lambda i: (i, 0)),
            pl.BlockSpec((d, d), lambda i: (0, 0)),
            pl.BlockSpec((1, d), lambda i: (0, 0)),
            pl.BlockSpec((d, fp), lambda i: (0, 0)),
            pl.BlockSpec((1, fp), lambda i: (0, 0)),
        ],
        out_specs=[
            pl.BlockSpec((blk, d), lambda i: (i, 0)),
            pl.BlockSpec((g, fp), lambda i: (0, 0)),
        ],
        out_shape=[
            jax.ShapeDtypeStruct((n, d), jnp.float32),
            jax.ShapeDtypeStruct((g, fp), jnp.float32),
        ],
    )(parts, x_cur, onehot, hw_t, hb, ww_t, wb)


def kernel(x, edge_index, batch, H1_w, H1_b, W1_w, W1_b, H2_w, H2_b, W2_w, W2_b):
    n, d = x.shape
    fp = W1_w.shape[0]
    g = 64
    e = edge_index.shape[1]
    npad = ((n + 1 + 255) // 256) * 256

    q = 32 * _CHUNK
    ep = ((e + q - 1) // q) * q
    pad = ep - e
    src = jnp.concatenate(
        [edge_index[0].astype(jnp.int32), jnp.zeros((pad,), jnp.int32)])
    dst = jnp.concatenate(
        [edge_index[1].astype(jnp.int32), jnp.full((pad,), n, jnp.int32)])
    src = src.reshape(-1, _CHUNK)
    dst = dst.reshape(-1, _CHUNK)

    onehot = (batch[:, None] == jnp.arange(g, dtype=batch.dtype)[None, :])
    onehot = onehot.astype(jnp.float32)
    h1t, w1t = H1_w.T, W1_w.T
    h2t, w2t = H2_w.T, W2_w.T
    h1b, w1b = H1_b.reshape(1, d), W1_b.reshape(1, fp)
    h2b, w2b = H2_b.reshape(1, d), W2_b.reshape(1, fp)

    blk = 400
    parts1 = _sc_aggregate(x, src, dst, npad)
    upd1, fp1 = _tc_dense(parts1, x, onehot, h1t, h1b, w1t, w1b, blk)
    parts2 = _sc_aggregate(upd1, src, dst, npad)
    _, fp2 = _tc_dense(parts2, upd1, onehot, h2t, h2b, w2t, w2b, blk)
    return fp1 + fp2


# speedup vs baseline: 1.3393x; 1.0536x over previous
import functools

import jax
import jax.numpy as jnp
from jax import lax
from jax.experimental import pallas as pl
from jax.experimental.pallas import tpu as pltpu
from jax.experimental.pallas import tpu_sc as plsc

_CHUNK = 128
_NW = 32


def _sc_aggregate(table, src2, dst2, npad):
    n, d = table.shape
    n_rows = src2.shape[0]
    total = n_rows // 16
    cpt0 = int(round(total * 0.72)) & ~1
    cpt1 = total - cpt0
    assert cpt0 > 0 and cpt1 > 0 and cpt1 % 2 == 0
    mesh = plsc.VectorSubcoreMesh(core_axis_name="c", subcore_axis_name="s")
    rpt = npad // 16

    @functools.partial(
        pl.kernel,
        mesh=mesh,
        out_type=jax.ShapeDtypeStruct((2, npad, d), jnp.float32),
        scratch_types=[
            pltpu.VMEM((_CHUNK,), jnp.int32),
            pltpu.VMEM((_CHUNK,), jnp.int32),
            pltpu.VMEM((_CHUNK,), jnp.int32),
            pltpu.VMEM((_CHUNK,), jnp.int32),
            pltpu.VMEM((_CHUNK, d), jnp.float32),
            pltpu.VMEM((_CHUNK, d), jnp.float32),
            pltpu.VMEM_SHARED((npad, d), jnp.float32),
            pltpu.SemaphoreType.DMA,
            pltpu.SemaphoreType.DMA,
            pltpu.SemaphoreType.DMA,
            pltpu.SemaphoreType.DMA,
        ],
    )
    def agg_kernel(table_hbm, src_hbm, dst_hbm, zeros_hbm, out_hbm,
                   src_a, dst_a, src_b, dst_b, rows_a, rows_b, acc_sh,
                   sem_a, sem_b, sem_ra, sem_rb):
        cid = lax.axis_index("c")
        sid = lax.axis_index("s")

        pltpu.sync_copy(zeros_hbm.at[pl.ds(sid * rpt, rpt)],
                        acc_sh.at[pl.ds(sid * rpt, rpt)])
        plsc.subcore_barrier()

        npairs = jnp.where(cid == 0, cpt0 // 2, cpt1 // 2)
        row0 = jnp.where(cid == 0, sid * cpt0, 16 * cpt0 + sid * cpt1)

        pltpu.async_copy(src_hbm.at[row0], src_a, sem_a)
        pltpu.async_copy(dst_hbm.at[row0], dst_a, sem_a)
        pltpu.async_copy(src_hbm.at[row0 + 1], src_b, sem_b)
        pltpu.async_copy(dst_hbm.at[row0 + 1], dst_b, sem_b)
        pltpu.make_async_copy(src_hbm.at[row0], src_a, sem_a).wait()
        pltpu.make_async_copy(dst_hbm.at[row0], dst_a, sem_a).wait()
        pltpu.async_copy(table_hbm.at[src_a], rows_a, sem_ra)

        def pair(i, carry):
            j = row0 + 2 * i
            not_last = i < npairs - 1
            pltpu.make_async_copy(table_hbm.at[src_a], rows_a, sem_ra).wait()
            pltpu.make_async_copy(src_hbm.at[j + 1], src_b, sem_b).wait()
            pltpu.make_async_copy(dst_hbm.at[j + 1], dst_b, sem_b).wait()
            pltpu.async_copy(table_hbm.at[src_b], rows_b, sem_rb)
            pltpu.sync_copy(rows_a, acc_sh.at[dst_a], add=True)

            @pl.when(not_last)
            def _pre_a():
                pltpu.async_copy(src_hbm.at[j + 2], src_a, sem_a)
                pltpu.async_copy(dst_hbm.at[j + 2], dst_a, sem_a)

            pltpu.make_async_copy(table_hbm.at[src_b], rows_b, sem_rb).wait()

            @pl.when(not_last)
            def _next_a():
                pltpu.make_async_copy(src_hbm.at[j + 2], src_a, sem_a).wait()
                pltpu.make_async_copy(dst_hbm.at[j + 2], dst_a, sem_a).wait()
                pltpu.async_copy(table_hbm.at[src_a], rows_a, sem_ra)

            pltpu.sync_copy(rows_b, acc_sh.at[dst_b], add=True)

            @pl.when(not_last)
            def _pre_b():
                pltpu.async_copy(src_hbm.at[j + 3], src_b, sem_b)
                pltpu.async_copy(dst_hbm.at[j + 3], dst_b, sem_b)

            return carry

        lax.fori_loop(0, npairs, pair, 0)
        plsc.subcore_barrier()

        pltpu.sync_copy(acc_sh.at[pl.ds(sid * rpt, rpt)],
                        out_hbm.at[cid, pl.ds(sid * rpt, rpt)])

    zeros = jnp.zeros((npad, d), jnp.float32)
    return agg_kernel(table, src2, dst2, zeros)


def _tc_dense(parts, x_cur, onehot, hw_t, hb, ww_t, wb, blk):
    n, d = x_cur.shape
    fp = ww_t.shape[1]
    g = onehot.shape[1]
    grid = n // blk
    def body(parts_ref, x_ref, oh_ref, hwt_ref, hb_ref, wwt_ref, wb_ref,
             upd_ref, fp_ref):
        agg = parts_ref[0] + parts_ref[1] + x_ref[...]
        pre = jnp.dot(agg, hwt_ref[...], preferred_element_type=jnp.float32)
        pre = pre + hb_ref[...]
        upd = 1.0 / (1.0 + jnp.exp(-pre))
        upd_ref[...] = upd
        logits = jnp.dot(upd, wwt_ref[...], preferred_element_type=jnp.float32)
        logits = logits + wb_ref[...]
        m = jnp.max(logits, axis=-1, keepdims=True)
        e = jnp.exp(logits - m)
        p = e / jnp.sum(e, axis=-1, keepdims=True)
        contrib = lax.dot_general(oh_ref[...], p, (((0,), (0,)), ((), ())),
                                  preferred_element_type=jnp.float32)

        @pl.when(pl.program_id(0) == 0)
        def _init():
            fp_ref[...] = contrib

        @pl.when(pl.program_id(0) != 0)
        def _acc():
            fp_ref[...] += contrib

    return pl.pallas_call(
        body,
        grid=(grid,),
        in_specs=[
            pl.BlockSpec((2, blk, d), lambda i: (0, i, 0)),
            pl.BlockSpec((blk, d), lambda i: (i, 0)),
            pl.BlockSpec((blk, g), lambda i: (i, 0)),
            pl.BlockSpec((d, d), lambda i: (0, 0)),
            pl.BlockSpec((1, d), lambda i: (0, 0)),
            pl.BlockSpec((d, fp), lambda i: (0, 0)),
            pl.BlockSpec((1, fp), lambda i: (0, 0)),
        ],
        out_specs=[
            pl.BlockSpec((blk, d), lambda i: (i, 0)),
            pl.BlockSpec((g, fp), lambda i: (0, 0)),
        ],
        out_shape=[
            jax.ShapeDtypeStruct((n, d), jnp.float32),
            jax.ShapeDtypeStruct((g, fp), jnp.float32),
        ],
    )(parts, x_cur, onehot, hw_t, hb, ww_t, wb)


def kernel(x, edge_index, batch, H1_w, H1_b, W1_w, W1_b, H2_w, H2_b, W2_w, W2_b):
    n, d = x.shape
    fp = W1_w.shape[0]
    g = 64
    e = edge_index.shape[1]
    npad = ((n + 1 + 255) // 256) * 256

    q = 32 * _CHUNK
    ep = ((e + q - 1) // q) * q
    pad = ep - e
    src = jnp.concatenate(
        [edge_index[0].astype(jnp.int32), jnp.zeros((pad,), jnp.int32)])
    dst = jnp.concatenate(
        [edge_index[1].astype(jnp.int32), jnp.full((pad,), n, jnp.int32)])
    src = src.reshape(-1, _CHUNK)
    dst = dst.reshape(-1, _CHUNK)

    onehot = (batch[:, None] == jnp.arange(g, dtype=batch.dtype)[None, :])
    onehot = onehot.astype(jnp.float32)
    h1t, w1t = H1_w.T, W1_w.T
    h2t, w2t = H2_w.T, W2_w.T
    h1b, w1b = H1_b.reshape(1, d), W1_b.reshape(1, fp)
    h2b, w2b = H2_b.reshape(1, d), W2_b.reshape(1, fp)

    blk = 400
    parts1 = _sc_aggregate(x, src, dst, npad)
    upd1, fp1 = _tc_dense(parts1, x, onehot, h1t, h1b, w1t, w1b, blk)
    parts2 = _sc_aggregate(upd1, src, dst, npad)
    _, fp2 = _tc_dense(parts2, upd1, onehot, h2t, h2b, w2t, w2b, blk)
    return fp1 + fp2
